# Initial kernel scaffold; baseline (speedup 1.0000x reference)
import jax
import jax.numpy as jnp
from jax.experimental import pallas as pl


def kernel(input_ids, wte, wpe):
    raise NotImplementedError("write your pallas kernel here")


# speedup vs baseline: 1.3980x; 1.3980x over previous
import functools

import jax
import jax.numpy as jnp
from jax import lax
from jax.experimental import pallas as pl
from jax.experimental.pallas import tpu as pltpu
from jax.experimental.pallas import tpu_sc as plsc

VOCAB = 50257
N_POS = 2048
N_EMBD = 768
BATCH = 4
SEQ = 2048

NTOK = BATCH * SEQ
NW = 32
TPW = NTOK // NW
CHUNK = 64
NCHUNK = TPW // CHUNK
LANES = 16
VPR = N_EMBD // LANES

_mesh = plsc.VectorSubcoreMesh(core_axis_name="c", subcore_axis_name="s")


@functools.partial(
    pl.kernel,
    mesh=_mesh,
    out_type=jax.ShapeDtypeStruct((NTOK, N_EMBD), jnp.float32),
    scratch_types=[
        pltpu.VMEM((TPW,), jnp.int32),
        pltpu.VMEM((CHUNK, N_EMBD), jnp.float32),
        pltpu.VMEM((CHUNK, N_EMBD), jnp.float32),
        pltpu.SemaphoreType.DMA,
    ],
)
def _embd_sc(ids_hbm, wte_hbm, wpe_hbm, out_hbm, idx_v, rows_v, wpe_v, sem):
    wid = lax.axis_index("s") * 2 + lax.axis_index("c")
    base = wid * TPW
    pos_base = lax.rem(base, SEQ)

    pltpu.sync_copy(ids_hbm.at[pl.ds(base, TPW)], idx_v)

    def chunk_body(c, carry):
        rbase = c * CHUNK
        gather = pltpu.async_copy(
            wte_hbm.at[idx_v.at[pl.ds(rbase, CHUNK)]], rows_v, sem)
        pltpu.sync_copy(wpe_hbm.at[pl.ds(pos_base + rbase, CHUNK)], wpe_v)
        gather.wait()

        def row_body(r, carry2):
            for i in range(VPR):
                sl = pl.ds(i * LANES, LANES)
                rows_v[r, sl] = rows_v[r, sl] + wpe_v[r, sl]
            return carry2

        lax.fori_loop(0, CHUNK, row_body, 0)
        pltpu.sync_copy(rows_v, out_hbm.at[pl.ds(base + rbase, CHUNK)])
        return carry

    lax.fori_loop(0, NCHUNK, chunk_body, 0)


@jax.jit
def kernel(input_ids, wte, wpe):
    ids = input_ids.reshape(NTOK).astype(jnp.int32)
    out = _embd_sc(ids, wte, wpe)
    return out.reshape(BATCH, SEQ, N_EMBD)


# speedup vs baseline: 1.5438x; 1.1043x over previous
import functools

import jax
import jax.numpy as jnp
from jax import lax
from jax.experimental import pallas as pl
from jax.experimental.pallas import tpu as pltpu
from jax.experimental.pallas import tpu_sc as plsc

VOCAB = 50257
N_POS = 2048
N_EMBD = 768
BATCH = 4
SEQ = 2048

NW = 32
PPW = SEQ // NW
SUB = 32
NSUB = BATCH * PPW // SUB
NBUF = 3
LANES = 16
VPR = N_EMBD // LANES

_mesh = plsc.VectorSubcoreMesh(core_axis_name="c", subcore_axis_name="s")


@functools.partial(
    pl.kernel,
    mesh=_mesh,
    out_type=jax.ShapeDtypeStruct((BATCH * SEQ, N_EMBD), jnp.float32),
    scratch_types=[
        pltpu.VMEM((BATCH, PPW), jnp.int32),
        pltpu.VMEM((PPW, N_EMBD), jnp.float32),
    ]
    + [pltpu.VMEM((SUB, N_EMBD), jnp.float32) for _ in range(NBUF)]
    + [pltpu.SemaphoreType.DMA for _ in range(1 + 2 * NBUF)],
)
def _embd_sc(ids_hbm, wte_hbm, wpe_hbm, out_hbm, idx_v, wpe_v,
             buf0, buf1, buf2, sem_wpe, sg0, sg1, sg2, ss0, ss1, ss2):
    bufs = (buf0, buf1, buf2)
    sg = (sg0, sg1, sg2)
    ss = (ss0, ss1, ss2)
    wid = lax.axis_index("s") * 2 + lax.axis_index("c")
    pos0 = wid * PPW

    wpe_dma = pltpu.async_copy(wpe_hbm.at[pl.ds(pos0, PPW)], wpe_v, sem_wpe)
    for b in range(BATCH):
        pltpu.sync_copy(ids_hbm.at[pl.ds(b * SEQ + pos0, PPW)], idx_v.at[b])

    def flat_base(j):
        return (j // 2) * SEQ + pos0 + (j % 2) * SUB

    def start_gather(j):
        k = j % NBUF
        idx = idx_v.at[j // 2, pl.ds((j % 2) * SUB, SUB)]
        return pltpu.async_copy(wte_hbm.at[idx], bufs[k], sg[k])

    gathers = [None] * NSUB
    stores = [None] * NSUB
    gathers[0] = start_gather(0)
    gathers[1] = start_gather(1)
    wpe_dma.wait()

    for j in range(NSUB):
        k = j % NBUF
        gathers[j].wait()
        prow = (j % 2) * SUB
        buf = bufs[k]

        def row_body(r, carry, buf=buf, prow=prow):
            for i in range(VPR):
                sl = pl.ds(i * LANES, LANES)
                plsc.addupdate(buf.at[r, sl], wpe_v[prow + r, sl])
            return carry

        lax.fori_loop(0, SUB, row_body, 0)
        stores[j] = pltpu.async_copy(
            buf, out_hbm.at[pl.ds(flat_base(j), SUB)], ss[k])
        if j + 2 < NSUB:
            if stores[j + 2 - NBUF] is not None:
                stores[j + 2 - NBUF].wait()
            gathers[j + 2] = start_gather(j + 2)

    for j in range(NSUB - NBUF, NSUB):
        stores[j].wait()


@jax.jit
def kernel(input_ids, wte, wpe):
    ids = input_ids.reshape(BATCH * SEQ).astype(jnp.int32)
    out = _embd_sc(ids, wte, wpe)
    return out.reshape(BATCH, SEQ, N_EMBD)
